# baseline (speedup 1.0000x reference)
import jax
import jax.numpy as jnp
from jax import lax
from jax.experimental import pallas as pl
from jax.experimental.pallas import tpu as pltpu
from jax.experimental.pallas import tpu_sc as plsc

VOCAB, DIM, B, C, NEG = 1_000_000, 64, 16384, 20, 20
L = 16
NW = 2 * 16
BPW = B // NW
NBS = 32
NH = 2
NB = NBS // NH
NCHUNK = BPW // NBS
RPH = NB * C
GROUPS = ((0, 128), (128, 128), (256, 64))
DK = DIM // L


def _sc_body(u_tab, v_tab, u_row, vp_row, vn_row, u_off, vp_off, vn_off,
             out_pos, out_neg,
             idx_u, idx_p, idx_n, cbu, cbp, cbn,
             rows_u, rows_p, rows_n, part_p, part_n, sem):
    cid = lax.axis_index("c")
    sid = lax.axis_index("s")
    wid = cid * 16 + sid
    iota = lax.iota(jnp.int32, L)
    ik = [iota + k * L for k in range(DK)]

    def f16(s):
        return jnp.full((L,), s, jnp.int32)

    def chunk_body(ch, carry):
        base = wid * BPW + ch * NBS
        pltpu.sync_copy(u_row.at[pl.ds(base, NBS)], idx_u)
        pltpu.sync_copy(vp_row.at[pl.ds(base * C, NBS * C)], idx_p)
        pltpu.sync_copy(vn_row.at[pl.ds(base * NEG, NBS * NEG)], idx_n)
        pltpu.sync_copy(u_off.at[pl.ds(base, NBS)], cbu)
        pltpu.sync_copy(vp_off.at[pl.ds(base * C, NBS * C)], cbp)
        pltpu.sync_copy(vn_off.at[pl.ds(base * NEG, NBS * NEG)], cbn)
        cpu = pltpu.async_copy(u_tab.at[idx_u], rows_u, sem)

        for h in range(NH):
            cps = []
            for so, ln in GROUPS:
                cps.append(pltpu.async_copy(
                    v_tab.at[idx_p.at[pl.ds(h * RPH + so, ln)]],
                    rows_p.at[pl.ds(so, ln)], sem))
                cps.append(pltpu.async_copy(
                    v_tab.at[idx_n.at[pl.ds(h * RPH + so, ln)]],
                    rows_n.at[pl.ds(so, ln)], sem))
            if h == 0:
                cps.append(cpu)
            for cp in cps:
                cp.wait()

            def b_body(b, carry2):
                gb = h * NB + b
                cb = plsc.load_gather(cbu, [f16(gb)])
                rb = f16(gb)
                u = [plsc.load_gather(rows_u, [rb, cb + ik[k]])
                     for k in range(DK)]
                s0 = h * RPH + b * C

                def gsum(rows, cbase, n):
                    r = f16(b * C)
                    cb0 = plsc.load_gather(cbase, [f16(s0)])
                    acc = [plsc.load_gather(rows, [r, cb0 + ik[k]])
                           for k in range(DK)]
                    for c in range(1, n):
                        r = f16(b * C + c)
                        cbc = plsc.load_gather(cbase, [f16(s0 + c)])
                        for k in range(DK):
                            acc[k] = acc[k] + plsc.load_gather(
                                rows, [r, cbc + ik[k]])
                    return acc

                accp = gsum(rows_p, cbp, C)
                tp = accp[0] * u[0]
                for k in range(1, DK):
                    tp = tp + accp[k] * u[k]
                part_p[gb, :] = tp
                accn = gsum(rows_n, cbn, NEG)
                tn = accn[0] * u[0]
                for k in range(1, DK):
                    tn = tn + accn[k] * u[k]
                part_n[gb, :] = tn
                return carry2

            lax.fori_loop(0, NB, b_body, 0)

        pltpu.sync_copy(part_p, out_pos.at[pl.ds(base, NBS)])
        pltpu.sync_copy(part_n, out_neg.at[pl.ds(base, NBS)])
        return carry

    lax.fori_loop(0, NCHUNK, chunk_body, 0)


def _finish_body(pos_ref, neg_ref, out_ref):
    sp = jnp.sum(pos_ref[...], axis=1, keepdims=True)
    sn = jnp.sum(neg_ref[...], axis=1, keepdims=True)

    def log_sigmoid(x):
        return jnp.minimum(x, 0.0) - jnp.log1p(jnp.exp(-jnp.abs(x)))

    out_ref[0, 0] = -jnp.sum(log_sigmoid(sp) + log_sigmoid(-sn)) / B


def kernel(u_table, v_table, u_pos, v_pos, v_neg):
    u2 = u_table.reshape(VOCAB // 2, 2 * DIM)
    v2 = v_table.reshape(VOCAB // 2, 2 * DIM)
    up = u_pos.astype(jnp.int32)
    vp = v_pos.astype(jnp.int32).reshape(B * C)
    vn = v_neg.astype(jnp.int32).reshape(B * NEG)
    u_row, u_off = up >> 1, (up & 1) << 6
    vp_row, vp_off = vp >> 1, (vp & 1) << 6
    vn_row, vn_off = vn >> 1, (vn & 1) << 6

    sc = pl.kernel(
        _sc_body,
        out_type=(jax.ShapeDtypeStruct((B, L), jnp.float32),
                  jax.ShapeDtypeStruct((B, L), jnp.float32)),
        mesh=plsc.VectorSubcoreMesh(core_axis_name="c", subcore_axis_name="s"),
        scratch_types=[
            pltpu.VMEM((NBS,), jnp.int32),
            pltpu.VMEM((NBS * C,), jnp.int32),
            pltpu.VMEM((NBS * NEG,), jnp.int32),
            pltpu.VMEM((NBS,), jnp.int32),
            pltpu.VMEM((NBS * C,), jnp.int32),
            pltpu.VMEM((NBS * NEG,), jnp.int32),
            pltpu.VMEM((NBS, 2 * DIM), jnp.float32),
            pltpu.VMEM((RPH, 2 * DIM), jnp.float32),
            pltpu.VMEM((RPH, 2 * DIM), jnp.float32),
            pltpu.VMEM((NBS, L), jnp.float32),
            pltpu.VMEM((NBS, L), jnp.float32),
            pltpu.SemaphoreType.DMA,
        ],
        compiler_params=pltpu.CompilerParams(needs_layout_passes=False),
    )
    part_pos, part_neg = sc(u2, v2, u_row, vp_row, vn_row,
                            u_off, vp_off, vn_off)

    loss = pl.pallas_call(
        _finish_body,
        out_shape=jax.ShapeDtypeStruct((1, 1), jnp.float32),
        out_specs=pl.BlockSpec(memory_space=pltpu.SMEM),
    )(part_pos, part_neg)
    return loss[0, 0]
